# baseline (device time: 204866 ns/iter reference)
import jax
import jax.numpy as jnp
from jax import lax
from jax.experimental import pallas as pl
from jax.experimental.pallas import tpu as pltpu

N_DEV = 8

RING2LOG = [0, 4, 7, 3, 2, 6, 5, 1]
CHORD = {0: 3, 3: 0, 1: 6, 6: 1, 2: 5, 5: 2, 4: 7, 7: 4}

TREES = {
    (0, 0): [(0, 1), (1, 2), (0, 3), (3, 4), (0, 7), (1, 6), (2, 5)],
    (0, 1): [(0, 7), (0, 3), (0, 1), (7, 6), (7, 4), (3, 2), (2, 5)],
    (1, 0): [(1, 2), (1, 0), (1, 6), (6, 5), (0, 7), (7, 4), (0, 3)],
    (1, 1): [(1, 0), (1, 6), (1, 2), (2, 3), (3, 4), (6, 7), (6, 5)],
    (2, 0): [(2, 3), (2, 1), (3, 0), (3, 4), (1, 6), (4, 7), (4, 5)],
    (2, 1): [(2, 3), (2, 1), (1, 0), (2, 5), (5, 6), (5, 4), (6, 7)],
    (3, 0): [(3, 2), (3, 4), (3, 0), (4, 7), (0, 1), (2, 5), (7, 6)],
    (3, 1): [(3, 4), (3, 2), (3, 0), (0, 7), (2, 1), (2, 5), (1, 6)],
    (4, 0): [(4, 5), (5, 2), (4, 7), (7, 0), (7, 6), (0, 3), (0, 1)],
    (4, 1): [(4, 3), (4, 7), (4, 5), (5, 6), (5, 2), (3, 0), (6, 1)],
    (5, 0): [(5, 6), (5, 4), (5, 2), (2, 1), (2, 3), (6, 7), (3, 0)],
    (5, 1): [(5, 6), (5, 2), (5, 4), (4, 7), (4, 3), (2, 1), (1, 0)],
    (6, 0): [(6, 5), (6, 1), (6, 7), (7, 0), (5, 4), (1, 2), (4, 3)],
    (6, 1): [(6, 7), (6, 5), (5, 4), (5, 2), (2, 3), (6, 1), (1, 0)],
    (7, 0): [(7, 0), (7, 4), (7, 6), (4, 5), (0, 1), (4, 3), (1, 2)],
    (7, 1): [(7, 4), (7, 6), (7, 0), (6, 1), (6, 5), (4, 3), (3, 2)],
}
SLOTS = [(o, half) for o in range(N_DEV) for half in range(2)]


def kernel(x, w_mat, scale_x, scale_w):
    m_per, k = x.shape
    _, n_per = w_mat.shape
    m_glob = N_DEV * m_per
    m_h = m_per // 2

    def body(x_ref, w_ref, sx_ref, sw_ref, out_ref, xg_ref, w8_ref,
             send_sems, recv_sems):
        my = lax.axis_index("i")

        barrier_sem = pltpu.get_barrier_semaphore()
        for r in range(N_DEV):
            me_log = RING2LOG[r]
            nbrs = {RING2LOG[(r + 1) % N_DEV], RING2LOG[(r - 1) % N_DEV],
                    RING2LOG[CHORD[r]]}

            @pl.when(my == me_log)
            def _(nbrs=nbrs):
                for nb in nbrs:
                    pl.semaphore_signal(
                        barrier_sem, inc=1,
                        device_id=(nb,), device_id_type=pl.DeviceIdType.MESH,
                    )
        pl.semaphore_wait(barrier_sem, 3)

        xg_ref[pl.ds(my * m_per, m_per), :] = (
            x_ref[...].astype(jnp.float8_e5m2))

        scale = sx_ref[0] * sw_ref[0]

        def store(row0, nrows):
            blk = xg_ref[pl.ds(row0, nrows), :]
            acc = jnp.dot(blk, w8_ref[...], preferred_element_type=jnp.float32)
            out_ref[pl.ds(row0, nrows), :] = jnp.maximum(acc * scale, 0.0)

        def make(si, arc_idx):
            o_ring, half = SLOTS[si]
            u, w = TREES[SLOTS[si]][arc_idx]
            row0 = RING2LOG[o_ring] * m_per + half * m_h
            child_idx = sum(
                1 for (u2, _) in TREES[SLOTS[si]][:arc_idx] if u2 == u)
            sl = xg_ref.at[pl.ds(row0, m_h), :]
            return pltpu.make_async_remote_copy(
                src_ref=sl, dst_ref=sl,
                send_sem=send_sems.at[si, child_idx],
                recv_sem=recv_sems.at[si],
                device_id=(RING2LOG[w],),
                device_id_type=pl.DeviceIdType.MESH,
            )

        descs = [[make(si, a) for a in range(N_DEV - 1)]
                 for si in range(len(SLOTS))]

        for si, (o_ring, half) in enumerate(SLOTS):
            root_log = RING2LOG[o_ring]

            @pl.when(my == root_log)
            def _(si=si, o_ring=o_ring):
                for a, (u, _) in enumerate(TREES[SLOTS[si]]):
                    if u == o_ring:
                        descs[si][a].start()

        w8_ref[...] = w_ref[...].astype(jnp.float8_e5m2)

        for si, (o_ring, half) in enumerate(SLOTS):
            tree = TREES[SLOTS[si]]
            for a, (u, w) in enumerate(tree):
                if u != o_ring:
                    @pl.when(my == RING2LOG[u])
                    def _(si=si, a=a):
                        descs[si][a].start()

                @pl.when(my == RING2LOG[w])
                def _(si=si, a=a):
                    descs[si][a].wait_recv()

            store(RING2LOG[o_ring] * m_per + half * m_h, m_h)

        for si, (o_ring, half) in enumerate(SLOTS):
            for a, (u, _) in enumerate(TREES[SLOTS[si]]):
                @pl.when(my == RING2LOG[u])
                def _(si=si, a=a):
                    descs[si][a].wait_send()

    return pl.pallas_call(
        body,
        out_shape=jax.ShapeDtypeStruct((m_glob, n_per), jnp.float32),
        in_specs=[
            pl.BlockSpec(memory_space=pltpu.VMEM),
            pl.BlockSpec(memory_space=pltpu.VMEM),
            pl.BlockSpec(memory_space=pltpu.SMEM),
            pl.BlockSpec(memory_space=pltpu.SMEM),
        ],
        out_specs=pl.BlockSpec(memory_space=pltpu.VMEM),
        scratch_shapes=[
            pltpu.VMEM((m_glob, k), jnp.float8_e5m2),
            pltpu.VMEM((k, n_per), jnp.float8_e5m2),
            pltpu.SemaphoreType.DMA((len(SLOTS), 3)),
            pltpu.SemaphoreType.DMA((len(SLOTS),)),
        ],
        compiler_params=pltpu.CompilerParams(collective_id=0),
    )(x, w_mat, scale_x, scale_w)


# device time: 86076 ns/iter; 2.3801x vs baseline; 2.3801x over previous
import jax
import jax.numpy as jnp
from jax import lax
from jax.experimental import pallas as pl
from jax.experimental.pallas import tpu as pltpu

N_DEV = 8

RING2LOG = [0, 4, 7, 3, 2, 6, 5, 1]
CHORD = {0: 3, 3: 0, 1: 6, 6: 1, 2: 5, 5: 2, 4: 7, 7: 4}

TREES = {
    (0, 0): [(0, 1), (1, 2), (0, 3), (3, 4), (0, 7), (1, 6), (2, 5)],
    (0, 1): [(0, 7), (0, 3), (0, 1), (7, 6), (7, 4), (3, 2), (2, 5)],
    (1, 0): [(1, 2), (1, 0), (1, 6), (6, 5), (0, 7), (7, 4), (0, 3)],
    (1, 1): [(1, 0), (1, 6), (1, 2), (2, 3), (3, 4), (6, 7), (6, 5)],
    (2, 0): [(2, 3), (2, 1), (3, 0), (3, 4), (1, 6), (4, 7), (4, 5)],
    (2, 1): [(2, 3), (2, 1), (1, 0), (2, 5), (5, 6), (5, 4), (6, 7)],
    (3, 0): [(3, 2), (3, 4), (3, 0), (4, 7), (0, 1), (2, 5), (7, 6)],
    (3, 1): [(3, 4), (3, 2), (3, 0), (0, 7), (2, 1), (2, 5), (1, 6)],
    (4, 0): [(4, 5), (5, 2), (4, 7), (7, 0), (7, 6), (0, 3), (0, 1)],
    (4, 1): [(4, 3), (4, 7), (4, 5), (5, 6), (5, 2), (3, 0), (6, 1)],
    (5, 0): [(5, 6), (5, 4), (5, 2), (2, 1), (2, 3), (6, 7), (3, 0)],
    (5, 1): [(5, 6), (5, 2), (5, 4), (4, 7), (4, 3), (2, 1), (1, 0)],
    (6, 0): [(6, 5), (6, 1), (6, 7), (7, 0), (5, 4), (1, 2), (4, 3)],
    (6, 1): [(6, 7), (6, 5), (5, 4), (5, 2), (2, 3), (6, 1), (1, 0)],
    (7, 0): [(7, 0), (7, 4), (7, 6), (4, 5), (0, 1), (4, 3), (1, 2)],
    (7, 1): [(7, 4), (7, 6), (7, 0), (6, 1), (6, 5), (4, 3), (3, 2)],
}
SLOTS = [(o, half) for o in range(N_DEV) for half in range(2)]


def kernel(x, w_mat, scale_x, scale_w):
    m_per, k = x.shape
    _, n_per = w_mat.shape
    m_glob = N_DEV * m_per
    m_h = m_per // 2

    def body(x_ref, w_ref, sx_ref, sw_ref, out_ref, xg_ref, w8_ref,
             send_sems, recv_sems):
        my = lax.axis_index("i")

        barrier_sem = pltpu.get_barrier_semaphore()
        for r in range(N_DEV):
            me_log = RING2LOG[r]
            nbrs = {RING2LOG[(r + 1) % N_DEV], RING2LOG[(r - 1) % N_DEV],
                    RING2LOG[CHORD[r]]}

            @pl.when(my == me_log)
            def _(nbrs=nbrs):
                for nb in nbrs:
                    pl.semaphore_signal(
                        barrier_sem, inc=1,
                        device_id=(nb,), device_id_type=pl.DeviceIdType.MESH,
                    )
        pl.semaphore_wait(barrier_sem, 3)

        xg_ref[pl.ds(my * m_per, m_per), :] = (
            x_ref[...].astype(jnp.float8_e5m2))

        scale = sx_ref[0] * sw_ref[0]

        def store(row0, nrows):
            blk = xg_ref[pl.ds(row0, nrows), :]
            acc = jnp.dot(blk, w8_ref[...], preferred_element_type=jnp.float32)
            out_ref[pl.ds(row0, nrows), :] = jnp.maximum(acc * scale, 0.0)

        def make(si, arc_idx):
            o_ring, half = SLOTS[si]
            u, w = TREES[SLOTS[si]][arc_idx]
            row0 = RING2LOG[o_ring] * m_per + half * m_h
            child_idx = sum(
                1 for (u2, _) in TREES[SLOTS[si]][:arc_idx] if u2 == u)
            sl = xg_ref.at[pl.ds(row0, m_h), :]
            return pltpu.make_async_remote_copy(
                src_ref=sl, dst_ref=sl,
                send_sem=send_sems.at[si, child_idx],
                recv_sem=recv_sems.at[si],
                device_id=(RING2LOG[w],),
                device_id_type=pl.DeviceIdType.MESH,
            )

        descs = [[make(si, a) for a in range(N_DEV - 1)]
                 for si in range(len(SLOTS))]

        for si, (o_ring, half) in enumerate(SLOTS):
            root_log = RING2LOG[o_ring]

            @pl.when(my == root_log)
            def _(si=si, o_ring=o_ring):
                for a, (u, _) in enumerate(TREES[SLOTS[si]]):
                    if u == o_ring:
                        descs[si][a].start()

        w8_ref[...] = w_ref[...].astype(jnp.float8_e5m2)
        store(my * m_per, m_per)

        def depths(si):
            o_ring, _ = SLOTS[si]
            d = {o_ring: 0}
            out = []
            for (u, w) in TREES[SLOTS[si]]:
                d[w] = d[u] + 1
                out.append(d[w])
            return out, d

        ARC_DEPTH = [depths(si)[0] for si in range(len(SLOTS))]
        NODE_DEPTH = [depths(si)[1] for si in range(len(SLOTS))]
        max_depth = max(max(ad) for ad in ARC_DEPTH)

        for r in range(1, max_depth + 1):
            for si in range(len(SLOTS)):
                for a, (u, w) in enumerate(TREES[SLOTS[si]]):
                    if ARC_DEPTH[si][a] == r:
                        @pl.when(my == RING2LOG[w])
                        def _(si=si, a=a):
                            descs[si][a].wait_recv()
                    elif ARC_DEPTH[si][a] == r + 1:
                        @pl.when(my == RING2LOG[u])
                        def _(si=si, a=a):
                            descs[si][a].start()
            for si, (o_ring, half) in enumerate(SLOTS):
                recvers = [w for w, d in NODE_DEPTH[si].items() if d == r]
                pred = my == RING2LOG[recvers[0]]
                for w in recvers[1:]:
                    pred = jnp.logical_or(pred, my == RING2LOG[w])

                @pl.when(pred)
                def _(si=si, o_ring=o_ring, half=half):
                    store(RING2LOG[o_ring] * m_per + half * m_h, m_h)

        for si, (o_ring, half) in enumerate(SLOTS):
            for a, (u, _) in enumerate(TREES[SLOTS[si]]):
                @pl.when(my == RING2LOG[u])
                def _(si=si, a=a):
                    descs[si][a].wait_send()

    return pl.pallas_call(
        body,
        out_shape=jax.ShapeDtypeStruct((m_glob, n_per), jnp.float32),
        in_specs=[
            pl.BlockSpec(memory_space=pltpu.VMEM),
            pl.BlockSpec(memory_space=pltpu.VMEM),
            pl.BlockSpec(memory_space=pltpu.SMEM),
            pl.BlockSpec(memory_space=pltpu.SMEM),
        ],
        out_specs=pl.BlockSpec(memory_space=pltpu.VMEM),
        scratch_shapes=[
            pltpu.VMEM((m_glob, k), jnp.float8_e5m2),
            pltpu.VMEM((k, n_per), jnp.float8_e5m2),
            pltpu.SemaphoreType.DMA((len(SLOTS), 3)),
            pltpu.SemaphoreType.DMA((len(SLOTS),)),
        ],
        compiler_params=pltpu.CompilerParams(collective_id=0),
    )(x, w_mat, scale_x, scale_w)


# device time: 84126 ns/iter; 2.4352x vs baseline; 1.0232x over previous
import jax
import jax.numpy as jnp
from jax import lax
from jax.experimental import pallas as pl
from jax.experimental.pallas import tpu as pltpu

N_DEV = 8

RING2LOG = [0, 4, 7, 3, 2, 6, 5, 1]
CHORD = {0: 3, 3: 0, 1: 6, 6: 1, 2: 5, 5: 2, 4: 7, 7: 4}

TREES = {
    (0, 0): [(0, 3), (0, 1), (0, 7), (7, 4), (7, 6), (3, 2), (4, 5)],
    (0, 1): [(0, 3), (0, 7), (3, 4), (0, 1), (1, 6), (1, 2), (2, 5)],
    (1, 0): [(1, 6), (1, 2), (2, 3), (6, 5), (1, 0), (6, 7), (7, 4)],
    (1, 1): [(1, 0), (1, 6), (0, 7), (1, 2), (2, 5), (2, 3), (3, 4)],
    (2, 0): [(2, 5), (5, 4), (4, 3), (2, 1), (1, 6), (1, 0), (0, 7)],
    (2, 1): [(2, 3), (2, 1), (2, 5), (3, 4), (4, 7), (5, 6), (3, 0)],
    (3, 0): [(3, 0), (3, 2), (3, 4), (0, 7), (7, 6), (2, 5), (0, 1)],
    (3, 1): [(3, 4), (3, 2), (3, 0), (4, 7), (4, 5), (2, 1), (5, 6)],
    (4, 0): [(4, 7), (4, 5), (4, 3), (7, 6), (3, 2), (3, 0), (0, 1)],
    (4, 1): [(4, 7), (4, 3), (4, 5), (3, 0), (3, 2), (5, 6), (6, 1)],
    (5, 0): [(5, 4), (5, 6), (6, 1), (5, 2), (4, 3), (6, 7), (7, 0)],
    (5, 1): [(5, 4), (5, 2), (5, 6), (2, 1), (6, 7), (1, 0), (2, 3)],
    (6, 0): [(6, 1), (6, 5), (6, 7), (7, 4), (5, 2), (1, 0), (0, 3)],
    (6, 1): [(6, 7), (6, 5), (5, 4), (5, 2), (2, 1), (7, 0), (2, 3)],
    (7, 0): [(7, 0), (7, 4), (7, 6), (0, 3), (4, 5), (6, 1), (5, 2)],
    (7, 1): [(7, 4), (7, 0), (7, 6), (0, 1), (0, 3), (6, 5), (1, 2)],
}
SLOTS = [(o, half) for o in range(N_DEV) for half in range(2)]


def kernel(x, w_mat, scale_x, scale_w):
    m_per, k = x.shape
    _, n_per = w_mat.shape
    m_glob = N_DEV * m_per
    m_h = m_per // 2

    def body(x_ref, w_ref, sx_ref, sw_ref, out_ref, xg_ref, w8_ref,
             send_sems, recv_sems):
        my = lax.axis_index("i")

        barrier_sem = pltpu.get_barrier_semaphore()
        for r in range(N_DEV):
            me_log = RING2LOG[r]
            nbrs = {RING2LOG[(r + 1) % N_DEV], RING2LOG[(r - 1) % N_DEV],
                    RING2LOG[CHORD[r]]}

            @pl.when(my == me_log)
            def _(nbrs=nbrs):
                for nb in nbrs:
                    pl.semaphore_signal(
                        barrier_sem, inc=1,
                        device_id=(nb,), device_id_type=pl.DeviceIdType.MESH,
                    )
        pl.semaphore_wait(barrier_sem, 3)

        xg_ref[pl.ds(my * m_per, m_per), :] = (
            x_ref[...].astype(jnp.float8_e5m2))

        scale = sx_ref[0] * sw_ref[0]

        def store(row0, nrows):
            blk = xg_ref[pl.ds(row0, nrows), :]
            acc = jnp.dot(blk, w8_ref[...], preferred_element_type=jnp.float32)
            out_ref[pl.ds(row0, nrows), :] = jnp.maximum(acc * scale, 0.0)

        def make(si, arc_idx):
            o_ring, half = SLOTS[si]
            u, w = TREES[SLOTS[si]][arc_idx]
            row0 = RING2LOG[o_ring] * m_per + half * m_h
            child_idx = sum(
                1 for (u2, _) in TREES[SLOTS[si]][:arc_idx] if u2 == u)
            sl = xg_ref.at[pl.ds(row0, m_h), :]
            return pltpu.make_async_remote_copy(
                src_ref=sl, dst_ref=sl,
                send_sem=send_sems.at[si, child_idx],
                recv_sem=recv_sems.at[si],
                device_id=(RING2LOG[w],),
                device_id_type=pl.DeviceIdType.MESH,
            )

        descs = [[make(si, a) for a in range(N_DEV - 1)]
                 for si in range(len(SLOTS))]

        for si, (o_ring, half) in enumerate(SLOTS):
            root_log = RING2LOG[o_ring]

            @pl.when(my == root_log)
            def _(si=si, o_ring=o_ring):
                for a, (u, _) in enumerate(TREES[SLOTS[si]]):
                    if u == o_ring:
                        descs[si][a].start()

        w8_ref[...] = w_ref[...].astype(jnp.float8_e5m2)
        store(my * m_per, m_per)

        def depths(si):
            o_ring, _ = SLOTS[si]
            d = {o_ring: 0}
            out = []
            for (u, w) in TREES[SLOTS[si]]:
                d[w] = d[u] + 1
                out.append(d[w])
            return out, d

        ARC_DEPTH = [depths(si)[0] for si in range(len(SLOTS))]
        NODE_DEPTH = [depths(si)[1] for si in range(len(SLOTS))]
        max_depth = max(max(ad) for ad in ARC_DEPTH)

        for r in range(1, max_depth + 1):
            for si in range(len(SLOTS)):
                for a, (u, w) in enumerate(TREES[SLOTS[si]]):
                    if ARC_DEPTH[si][a] == r:
                        @pl.when(my == RING2LOG[w])
                        def _(si=si, a=a):
                            descs[si][a].wait_recv()
                    elif ARC_DEPTH[si][a] == r + 1:
                        @pl.when(my == RING2LOG[u])
                        def _(si=si, a=a):
                            descs[si][a].start()
            for si, (o_ring, half) in enumerate(SLOTS):
                recvers = [w for w, d in NODE_DEPTH[si].items() if d == r]
                pred = my == RING2LOG[recvers[0]]
                for w in recvers[1:]:
                    pred = jnp.logical_or(pred, my == RING2LOG[w])

                @pl.when(pred)
                def _(si=si, o_ring=o_ring, half=half):
                    store(RING2LOG[o_ring] * m_per + half * m_h, m_h)

        for si, (o_ring, half) in enumerate(SLOTS):
            for a, (u, _) in enumerate(TREES[SLOTS[si]]):
                @pl.when(my == RING2LOG[u])
                def _(si=si, a=a):
                    descs[si][a].wait_send()

    return pl.pallas_call(
        body,
        out_shape=jax.ShapeDtypeStruct((m_glob, n_per), jnp.float32),
        in_specs=[
            pl.BlockSpec(memory_space=pltpu.VMEM),
            pl.BlockSpec(memory_space=pltpu.VMEM),
            pl.BlockSpec(memory_space=pltpu.SMEM),
            pl.BlockSpec(memory_space=pltpu.SMEM),
        ],
        out_specs=pl.BlockSpec(memory_space=pltpu.VMEM),
        scratch_shapes=[
            pltpu.VMEM((m_glob, k), jnp.float8_e5m2),
            pltpu.VMEM((k, n_per), jnp.float8_e5m2),
            pltpu.SemaphoreType.DMA((len(SLOTS), 3)),
            pltpu.SemaphoreType.DMA((len(SLOTS),)),
        ],
        compiler_params=pltpu.CompilerParams(collective_id=0),
    )(x, w_mat, scale_x, scale_w)


# device time: 74878 ns/iter; 2.7360x vs baseline; 1.1235x over previous
import jax
import jax.numpy as jnp
from jax import lax
from jax.experimental import pallas as pl
from jax.experimental.pallas import tpu as pltpu

N_DEV = 8

RING2LOG = [0, 4, 7, 3, 2, 6, 5, 1]
CHORD = {0: 3, 3: 0, 1: 6, 6: 1, 2: 5, 5: 2, 4: 7, 7: 4}

TREES = {
    (0, 0): [(0, 3), (0, 1), (0, 7), (7, 4), (7, 6), (3, 2), (4, 5)],
    (0, 1): [(0, 3), (0, 7), (3, 4), (0, 1), (1, 6), (1, 2), (2, 5)],
    (1, 0): [(1, 6), (1, 2), (2, 3), (6, 5), (1, 0), (6, 7), (7, 4)],
    (1, 1): [(1, 0), (1, 6), (0, 7), (1, 2), (2, 5), (2, 3), (3, 4)],
    (2, 0): [(2, 5), (5, 4), (4, 3), (2, 1), (1, 6), (1, 0), (0, 7)],
    (2, 1): [(2, 3), (2, 1), (2, 5), (3, 4), (4, 7), (5, 6), (3, 0)],
    (3, 0): [(3, 0), (3, 2), (3, 4), (0, 7), (7, 6), (2, 5), (0, 1)],
    (3, 1): [(3, 4), (3, 2), (3, 0), (4, 7), (4, 5), (2, 1), (5, 6)],
    (4, 0): [(4, 7), (4, 5), (4, 3), (7, 6), (3, 2), (3, 0), (0, 1)],
    (4, 1): [(4, 7), (4, 3), (4, 5), (3, 0), (3, 2), (5, 6), (6, 1)],
    (5, 0): [(5, 4), (5, 6), (6, 1), (5, 2), (4, 3), (6, 7), (7, 0)],
    (5, 1): [(5, 4), (5, 2), (5, 6), (2, 1), (6, 7), (1, 0), (2, 3)],
    (6, 0): [(6, 1), (6, 5), (6, 7), (7, 4), (5, 2), (1, 0), (0, 3)],
    (6, 1): [(6, 7), (6, 5), (5, 4), (5, 2), (2, 1), (7, 0), (2, 3)],
    (7, 0): [(7, 0), (7, 4), (7, 6), (0, 3), (4, 5), (6, 1), (5, 2)],
    (7, 1): [(7, 4), (7, 0), (7, 6), (0, 1), (0, 3), (6, 5), (1, 2)],
}
SLOTS = [(o, half) for o in range(N_DEV) for half in range(2)]
SLOT_ORDER = [0, 8, 2, 3, 4, 5, 6, 7, 1, 15, 10, 11, 12, 13, 14, 9]


def kernel(x, w_mat, scale_x, scale_w):
    m_per, k = x.shape
    _, n_per = w_mat.shape
    m_glob = N_DEV * m_per
    m_h = m_per // 2

    def body(x_ref, w_ref, sx_ref, sw_ref, out_ref, xg_ref, w8_ref,
             send_sems, recv_sems):
        my = lax.axis_index("i")

        barrier_sem = pltpu.get_barrier_semaphore()
        for r in range(N_DEV):
            me_log = RING2LOG[r]
            nbrs = {RING2LOG[(r + 1) % N_DEV], RING2LOG[(r - 1) % N_DEV],
                    RING2LOG[CHORD[r]]}

            @pl.when(my == me_log)
            def _(nbrs=nbrs):
                for nb in nbrs:
                    pl.semaphore_signal(
                        barrier_sem, inc=1,
                        device_id=(nb,), device_id_type=pl.DeviceIdType.MESH,
                    )
        pl.semaphore_wait(barrier_sem, 3)

        xg_ref[pl.ds(my * m_per, m_per), :] = (
            x_ref[...].astype(jnp.float8_e5m2))

        scale = sx_ref[0] * sw_ref[0]

        def store(row0, nrows):
            blk = xg_ref[pl.ds(row0, nrows), :]
            acc = jnp.dot(blk, w8_ref[...], preferred_element_type=jnp.float32)
            out_ref[pl.ds(row0, nrows), :] = jnp.maximum(acc * scale, 0.0)

        def make(si, arc_idx):
            o_ring, half = SLOTS[si]
            u, w = TREES[SLOTS[si]][arc_idx]
            row0 = RING2LOG[o_ring] * m_per + half * m_h
            child_idx = sum(
                1 for (u2, _) in TREES[SLOTS[si]][:arc_idx] if u2 == u)
            sl = xg_ref.at[pl.ds(row0, m_h), :]
            return pltpu.make_async_remote_copy(
                src_ref=sl, dst_ref=sl,
                send_sem=send_sems.at[si, child_idx],
                recv_sem=recv_sems.at[si],
                device_id=(RING2LOG[w],),
                device_id_type=pl.DeviceIdType.MESH,
            )

        descs = [[make(si, a) for a in range(N_DEV - 1)]
                 for si in range(len(SLOTS))]

        for si in SLOT_ORDER:
            o_ring, half = SLOTS[si]
            root_log = RING2LOG[o_ring]

            @pl.when(my == root_log)
            def _(si=si, o_ring=o_ring):
                for a, (u, _) in enumerate(TREES[SLOTS[si]]):
                    if u == o_ring:
                        descs[si][a].start()

        w8_ref[...] = w_ref[...].astype(jnp.float8_e5m2)
        store(my * m_per, m_per)

        def depths(si):
            o_ring, _ = SLOTS[si]
            d = {o_ring: 0}
            out = []
            for (u, w) in TREES[SLOTS[si]]:
                d[w] = d[u] + 1
                out.append(d[w])
            return out, d

        ARC_DEPTH = [depths(si)[0] for si in range(len(SLOTS))]
        NODE_DEPTH = [depths(si)[1] for si in range(len(SLOTS))]
        max_depth = max(max(ad) for ad in ARC_DEPTH)

        for r in range(1, max_depth + 1):
            for si in SLOT_ORDER:
                for a, (u, w) in enumerate(TREES[SLOTS[si]]):
                    if ARC_DEPTH[si][a] == r:
                        @pl.when(my == RING2LOG[w])
                        def _(si=si, a=a):
                            descs[si][a].wait_recv()
                    elif ARC_DEPTH[si][a] == r + 1:
                        @pl.when(my == RING2LOG[u])
                        def _(si=si, a=a):
                            descs[si][a].start()
            for si in SLOT_ORDER:
                o_ring, half = SLOTS[si]
                recvers = [w for w, d in NODE_DEPTH[si].items() if d == r]
                pred = my == RING2LOG[recvers[0]]
                for w in recvers[1:]:
                    pred = jnp.logical_or(pred, my == RING2LOG[w])

                @pl.when(pred)
                def _(si=si, o_ring=o_ring, half=half):
                    store(RING2LOG[o_ring] * m_per + half * m_h, m_h)

        for si, (o_ring, half) in enumerate(SLOTS):
            for a, (u, _) in enumerate(TREES[SLOTS[si]]):
                @pl.when(my == RING2LOG[u])
                def _(si=si, a=a):
                    descs[si][a].wait_send()

    return pl.pallas_call(
        body,
        out_shape=jax.ShapeDtypeStruct((m_glob, n_per), jnp.float32),
        in_specs=[
            pl.BlockSpec(memory_space=pltpu.VMEM),
            pl.BlockSpec(memory_space=pltpu.VMEM),
            pl.BlockSpec(memory_space=pltpu.SMEM),
            pl.BlockSpec(memory_space=pltpu.SMEM),
        ],
        out_specs=pl.BlockSpec(memory_space=pltpu.VMEM),
        scratch_shapes=[
            pltpu.VMEM((m_glob, k), jnp.float8_e5m2),
            pltpu.VMEM((k, n_per), jnp.float8_e5m2),
            pltpu.SemaphoreType.DMA((len(SLOTS), 3)),
            pltpu.SemaphoreType.DMA((len(SLOTS),)),
        ],
        compiler_params=pltpu.CompilerParams(collective_id=0),
    )(x, w_mat, scale_x, scale_w)
